# baseline (speedup 1.0000x reference)
import jax
import jax.numpy as jnp
from jax import lax
from jax.experimental import pallas as pl
from jax.experimental.pallas import tpu as pltpu
from jax.experimental.pallas import tpu_sc as plsc

_B = 4096
_L = 50
_HID = 64
_EPS = 1e-05
_N = _B * _L
_NC = 2
_NS = 16
_NW = _NC * _NS
_BPW = _B // _NW
_IDC = 8
_CROWS = _IDC * _L


def _sc_gather_body(ids_hbm, emb_hbm, gath_hbm, ids_v, sem):
    wid = lax.axis_index("s") * _NC + lax.axis_index("c")
    b0 = wid * _BPW

    for c in range(_BPW // _IDC):
        pltpu.sync_copy(ids_hbm.at[pl.ds(b0 + c * _IDC, _IDC)], ids_v)
        row0 = (b0 + c * _IDC) * _L

        def row_body(r, carry, row0=row0):
            bb = r // _L
            ll = r - bb * _L
            i = ids_v[bb, pl.ds(ll, 16)][0]
            pltpu.async_copy(emb_hbm.at[i], gath_hbm.at[row0 + r], sem)
            return carry

        lax.fori_loop(0, _CROWS, row_body, 0)

        pltpu.make_async_copy(
            gath_hbm.at[pl.ds(row0, _CROWS)],
            gath_hbm.at[pl.ds(row0, _CROWS)],
            sem,
        ).wait()


@jax.jit
def _sc_gather(ids, emb):
    mesh = plsc.VectorSubcoreMesh(
        core_axis_name="c", subcore_axis_name="s",
        num_cores=_NC, num_subcores=_NS,
    )
    return pl.kernel(
        _sc_gather_body,
        out_type=jax.ShapeDtypeStruct((_N, _HID), jnp.float32),
        mesh=mesh,
        scratch_types=[
            pltpu.VMEM((_IDC, _L), jnp.int32),
            pltpu.SemaphoreType.DMA,
        ],
        compiler_params=pltpu.CompilerParams(use_tc_tiling_on_sc=True),
    )(ids, emb)


_TCB = 16


def _tc_ln_body(gath_ref, mask_ref, w_ref, b_ref, out_ref):
    x = gath_ref[...]
    mu = jnp.mean(x, axis=1, keepdims=True)
    xc = x - mu
    var = jnp.mean(xc * xc, axis=1, keepdims=True)
    o = xc * lax.rsqrt(var + _EPS) * w_ref[...] + b_ref[...]
    m = mask_ref[...]
    out_ref[...] = o.reshape(_TCB, _L, _HID) * m[:, :, None]


@jax.jit
def _tc_ln(gath, mask, w, b):
    return pl.pallas_call(
        _tc_ln_body,
        grid=(_B // _TCB,),
        in_specs=[
            pl.BlockSpec((_TCB * _L, _HID), lambda i: (i, 0)),
            pl.BlockSpec((_TCB, _L), lambda i: (i, 0)),
            pl.BlockSpec((_HID,), lambda i: (0,)),
            pl.BlockSpec((_HID,), lambda i: (0,)),
        ],
        out_specs=pl.BlockSpec((_TCB, _L, _HID), lambda i: (i, 0, 0)),
        out_shape=jax.ShapeDtypeStruct((_B, _L, _HID), jnp.float32),
    )(gath, mask, w, b)


def kernel(input_ids, attention_mask, word_embeddings, ln_weight, ln_bias):
    ids = input_ids.astype(jnp.int32)
    gath = _sc_gather(ids, word_embeddings)
    return _tc_ln(gath, attention_mask.astype(jnp.float32),
                  ln_weight, ln_bias)


# speedup vs baseline: 3.5315x; 3.5315x over previous
import jax
import jax.numpy as jnp
from jax import lax
from jax.experimental import pallas as pl
from jax.experimental.pallas import tpu as pltpu
from jax.experimental.pallas import tpu_sc as plsc

_B = 4096
_L = 50
_HID = 64
_WID = 128
_EPS = 1e-05
_N = _B * _L
_V = 1000000
_NC = 2
_NS = 16
_NW = _NC * _NS
_BPW = _B // _NW
_CB = 16
_NCHUNK = _BPW // _CB
_CROWS = _CB * _L

_VBLK = 4096


def _widen_body(emb_ref, out_ref):
    x = emb_ref[...]
    out_ref[...] = jnp.concatenate([x, jnp.zeros_like(x)], axis=-1)


@jax.jit
def _tc_widen(emb):
    return pl.pallas_call(
        _widen_body,
        grid=(_V // _VBLK,),
        in_specs=[pl.BlockSpec((_VBLK, _HID), lambda i: (i, 0))],
        out_specs=pl.BlockSpec((_VBLK, _WID), lambda i: (i, 0)),
        out_shape=jax.ShapeDtypeStruct((_V, _WID), jnp.float32),
    )(emb)


def _sc_gather_body(ids_hbm, emb_hbm, gath_hbm, idx_v, rows_v, sem):
    wid = lax.axis_index("s") * _NC + lax.axis_index("c")
    b0 = wid * _BPW

    pltpu.sync_copy(ids_hbm.at[pl.ds(b0, _BPW)], idx_v)

    for c in range(_NCHUNK):
        copies = [
            pltpu.async_copy(
                emb_hbm.at[idx_v.at[c * _CB + bb]],
                rows_v.at[bb],
                sem,
            )
            for bb in range(_CB)
        ]
        for cp in copies:
            cp.wait()
        pltpu.sync_copy(rows_v, gath_hbm.at[pl.ds(b0 + c * _CB, _CB)])


@jax.jit
def _sc_gather(ids, emb128):
    mesh = plsc.VectorSubcoreMesh(
        core_axis_name="c", subcore_axis_name="s",
        num_cores=_NC, num_subcores=_NS,
    )
    return pl.kernel(
        _sc_gather_body,
        out_type=jax.ShapeDtypeStruct((_B, _L, _WID), jnp.float32),
        mesh=mesh,
        scratch_types=[
            pltpu.VMEM((_BPW, _L), jnp.int32),
            pltpu.VMEM((_CB, _L, _WID), jnp.float32),
            pltpu.SemaphoreType.DMA,
        ],
        compiler_params=pltpu.CompilerParams(use_tc_tiling_on_sc=True),
    )(ids, emb128)


_TCB = 16


def _tc_ln_body(gath_ref, mask_ref, w_ref, b_ref, out_ref):
    x = gath_ref[:, :, :_HID]
    mu = jnp.mean(x, axis=-1, keepdims=True)
    xc = x - mu
    var = jnp.mean(xc * xc, axis=-1, keepdims=True)
    o = xc * lax.rsqrt(var + _EPS) * w_ref[...] + b_ref[...]
    m = mask_ref[...]
    out_ref[...] = o * m[:, :, None]


@jax.jit
def _tc_ln(gath, mask, w, b):
    return pl.pallas_call(
        _tc_ln_body,
        grid=(_B // _TCB,),
        in_specs=[
            pl.BlockSpec((_TCB, _L, _WID), lambda i: (i, 0, 0)),
            pl.BlockSpec((_TCB, _L), lambda i: (i, 0)),
            pl.BlockSpec((_HID,), lambda i: (0,)),
            pl.BlockSpec((_HID,), lambda i: (0,)),
        ],
        out_specs=pl.BlockSpec((_TCB, _L, _HID), lambda i: (i, 0, 0)),
        out_shape=jax.ShapeDtypeStruct((_B, _L, _HID), jnp.float32),
    )(gath, mask, w, b)


def kernel(input_ids, attention_mask, word_embeddings, ln_weight, ln_bias):
    ids = input_ids.astype(jnp.int32)
    emb128 = _tc_widen(word_embeddings)
    gath = _sc_gather(ids, emb128)
    return _tc_ln(gath, attention_mask.astype(jnp.float32),
                  ln_weight, ln_bias)


# speedup vs baseline: 3.9834x; 1.1280x over previous
import jax
import jax.numpy as jnp
from jax import lax
from jax.experimental import pallas as pl
from jax.experimental.pallas import tpu as pltpu
from jax.experimental.pallas import tpu_sc as plsc

_B = 4096
_L = 50
_HID = 64
_WID = 128
_EPS = 1e-05
_N = _B * _L
_NC = 2
_NS = 16
_NW = _NC * _NS
_PER_W = _N // _NW
_GLEN = 128
_NGRP = _PER_W // _GLEN
_GPC = 8
_CROWS = _GPC * _GLEN
_NCHUNK = _NGRP // _GPC


def _sc_gather_body(idsf_hbm, emb_hbm, gath_hbm, idx_v, rows_v, sem):
    wid = lax.axis_index("s") * _NC + lax.axis_index("c")
    r0 = wid * _PER_W

    pltpu.sync_copy(idsf_hbm.at[pl.ds(wid * _NGRP, _NGRP)], idx_v)

    for c in range((_NGRP + _GPC - 1) // _GPC):
        g0 = c * _GPC
        ng = min(_GPC, _NGRP - g0)
        copies = [
            pltpu.async_copy(
                emb_hbm.at[idx_v.at[g0 + g]],
                rows_v.at[pl.ds(g * _GLEN, _GLEN)],
                sem,
            )
            for g in range(ng)
        ]
        for cp in copies:
            cp.wait()
        pltpu.sync_copy(
            rows_v.at[pl.ds(0, ng * _GLEN)],
            gath_hbm.at[pl.ds(r0 + g0 * _GLEN, ng * _GLEN),
                        pl.ds(0, _HID)],
        )


@jax.jit
def _sc_gather(idsf, emb):
    mesh = plsc.VectorSubcoreMesh(
        core_axis_name="c", subcore_axis_name="s",
        num_cores=_NC, num_subcores=_NS,
    )
    return pl.kernel(
        _sc_gather_body,
        out_type=jax.ShapeDtypeStruct((_N, _WID), jnp.float32),
        mesh=mesh,
        scratch_types=[
            pltpu.VMEM((_NGRP, _GLEN), jnp.int32),
            pltpu.VMEM((_CROWS, _HID), jnp.float32),
            pltpu.SemaphoreType.DMA,
        ],
        compiler_params=pltpu.CompilerParams(use_tc_tiling_on_sc=False),
    )(idsf, emb)


_TCB = 16


def _tc_ln_body(gath_ref, mask_ref, w_ref, b_ref, out_ref):
    x = gath_ref[:, :_HID]
    mu = jnp.mean(x, axis=1, keepdims=True)
    xc = x - mu
    var = jnp.mean(xc * xc, axis=1, keepdims=True)
    o = xc * lax.rsqrt(var + _EPS) * w_ref[...] + b_ref[...]
    m = mask_ref[...]
    out_ref[...] = o.reshape(_TCB, _L, _HID) * m[:, :, None]


@jax.jit
def _tc_ln(gath, mask, w, b):
    return pl.pallas_call(
        _tc_ln_body,
        grid=(_B // _TCB,),
        in_specs=[
            pl.BlockSpec((_TCB * _L, _WID), lambda i: (i, 0)),
            pl.BlockSpec((_TCB, _L), lambda i: (i, 0)),
            pl.BlockSpec((_HID,), lambda i: (0,)),
            pl.BlockSpec((_HID,), lambda i: (0,)),
        ],
        out_specs=pl.BlockSpec((_TCB, _L, _HID), lambda i: (i, 0, 0)),
        out_shape=jax.ShapeDtypeStruct((_B, _L, _HID), jnp.float32),
    )(gath, mask, w, b)


def kernel(input_ids, attention_mask, word_embeddings, ln_weight, ln_bias):
    idsf = input_ids.astype(jnp.int32).reshape(_N // _WID, _WID)
    gath = _sc_gather(idsf, word_embeddings)
    return _tc_ln(gath, attention_mask.astype(jnp.float32),
                  ln_weight, ln_bias)
